# baseline (speedup 1.0000x reference)
import jax
import jax.numpy as jnp
from jax import lax
from jax.experimental import pallas as pl
from jax.experimental.pallas import tpu as pltpu
from jax.experimental.pallas import tpu_sc as plsc

NC = 2
NS = 16
NW = NC * NS

MAXLEN = 200
D = 32
BATCH = 4096
BPW = BATCH // NW
TB = 4
N_CHUNKS = MAXLEN // TB
GROUPS = BPW // 16


def _body(xt_hbm, tok_hbm, pos_hbm, out_hbm, idx_v, rows0, rows1,
          buf0, buf1, pos_v, sem_g0, sem_g1, sem_s0, sem_s1):
    wid = lax.axis_index("s") * NC + lax.axis_index("c")
    b0 = wid * BPW

    rows = (rows0, rows1)
    bufs = (buf0, buf1)
    sem_g = (sem_g0, sem_g1)
    sem_s = (sem_s0, sem_s1)

    pltpu.sync_copy(xt_hbm.at[:, pl.ds(b0, BPW)], idx_v)
    pltpu.sync_copy(pos_hbm, pos_v)

    iota = lax.iota(jnp.int32, 16)
    zeros = iota * 0

    def fire_gathers(c, buf_ix):
        for tt in range(TB):
            pltpu.async_copy(
                tok_hbm.at[idx_v.at[c * TB + tt]],
                rows[buf_ix].at[pl.ds(tt * BPW, BPW)], sem_g[buf_ix])

    def drain_gathers(c, buf_ix):
        for tt in range(TB):
            pltpu.make_async_copy(
                tok_hbm.at[idx_v.at[c * TB + tt]],
                rows[buf_ix].at[pl.ds(tt * BPW, BPW)], sem_g[buf_ix]).wait()

    def transpose_add(c, buf_ix):
        src = rows[buf_ix]
        dst = bufs[buf_ix]

        def t_loop(tt, _):
            rbase = tt * BPW
            for d in range(D):
                pb = plsc.load_gather(pos_v, [zeros + (c * TB + tt),
                                              zeros + d])
                for g in range(GROUPS):
                    v = plsc.load_gather(
                        src, [rbase + g * 16 + iota, zeros + d])
                    dst[tt, d, pl.ds(g * 16, 16)] = v + pb
            return 0

        lax.fori_loop(0, TB, t_loop, 0)

    fire_gathers(0, 0)

    def pair_body(j, _):
        for par in range(2):
            c = 2 * j + par
            drain_gathers(c, par)

            @pl.when(c + 1 < N_CHUNKS)
            def _():
                fire_gathers(c + 1, 1 - par)

            @pl.when(c >= 2)
            def _():
                pltpu.make_async_copy(
                    bufs[par],
                    out_hbm.at[pl.ds((c - 2) * TB, TB), :, pl.ds(b0, BPW)],
                    sem_s[par]).wait()

            transpose_add(c, par)
            pltpu.async_copy(
                bufs[par], out_hbm.at[pl.ds(c * TB, TB), :, pl.ds(b0, BPW)],
                sem_s[par])
        return 0

    lax.fori_loop(0, N_CHUNKS // 2, pair_body, 0)

    for par in range(2):
        c = N_CHUNKS - 2 + par
        pltpu.make_async_copy(
            bufs[par], out_hbm.at[pl.ds(c * TB, TB), :, pl.ds(b0, BPW)],
            sem_s[par]).wait()


@jax.jit
def _embed(xt, token_table, pos_table):
    mesh = plsc.VectorSubcoreMesh(core_axis_name="c", subcore_axis_name="s")
    return pl.kernel(
        _body,
        out_type=jax.ShapeDtypeStruct((MAXLEN, D, BATCH), jnp.float32),
        mesh=mesh,
        scratch_types=[
            pltpu.VMEM((MAXLEN, BPW), jnp.int32),
            pltpu.VMEM((TB * BPW, D), jnp.float32),
            pltpu.VMEM((TB * BPW, D), jnp.float32),
            pltpu.VMEM((TB, D, BPW), jnp.float32),
            pltpu.VMEM((TB, D, BPW), jnp.float32),
            pltpu.VMEM((MAXLEN, D), jnp.float32),
            pltpu.SemaphoreType.DMA,
            pltpu.SemaphoreType.DMA,
            pltpu.SemaphoreType.DMA,
            pltpu.SemaphoreType.DMA,
        ],
        compiler_params=pltpu.CompilerParams(use_tc_tiling_on_sc=False,
                                             needs_layout_passes=False),
    )(xt, token_table, pos_table)


def kernel(x, token_table, pos_table):
    xt = jnp.swapaxes(x, 0, 1).astype(jnp.int32)
    out_tdb = _embed(xt, token_table, pos_table)
    return jnp.transpose(out_tdb, (2, 0, 1))


# speedup vs baseline: 1.6545x; 1.6545x over previous
import jax
import jax.numpy as jnp
from jax import lax
from jax.experimental import pallas as pl
from jax.experimental.pallas import tpu as pltpu
from jax.experimental.pallas import tpu_sc as plsc

NC = 2
NS = 16
NW = NC * NS

VOCAB = 100000
MAXLEN = 200
D = 32
BATCH = 4096
GROUPS = BATCH // 16


def _body(xt_hbm, tokT_hbm, pos_hbm, out_hbm, row_v, idx0, idx1,
          buf0, buf1, pos_v, sem_i0, sem_i1, sem_s0, sem_s1):
    d = lax.axis_index("s") * NC + lax.axis_index("c")

    idx = (idx0, idx1)
    bufs = (buf0, buf1)
    sem_i = (sem_i0, sem_i1)
    sem_s = (sem_s0, sem_s1)

    pltpu.sync_copy(tokT_hbm.at[d], row_v)
    pltpu.sync_copy(pos_hbm, pos_v)

    zeros = lax.iota(jnp.int32, 16) * 0

    pltpu.async_copy(xt_hbm.at[0], idx0, sem_i0)

    def pair_body(j, _):
        for par in range(2):
            t = 2 * j + par
            pltpu.make_async_copy(xt_hbm.at[t], idx[par], sem_i[par]).wait()

            @pl.when(t + 1 < MAXLEN)
            def _():
                pltpu.async_copy(xt_hbm.at[t + 1], idx[1 - par],
                                 sem_i[1 - par])

            pb = plsc.load_gather(pos_v, [zeros + t, zeros + d])

            @pl.when(t >= 2)
            def _():
                pltpu.make_async_copy(bufs[par], out_hbm.at[t - 2, d],
                                      sem_s[par]).wait()

            src = idx[par]
            dst = bufs[par]
            for g in range(GROUPS):
                iv = src[pl.ds(g * 16, 16)]
                v = plsc.load_gather(row_v, [iv])
                dst[pl.ds(g * 16, 16)] = v + pb

            pltpu.async_copy(bufs[par], out_hbm.at[t, d], sem_s[par])
        return 0

    lax.fori_loop(0, MAXLEN // 2, pair_body, 0)

    for par in range(2):
        t = MAXLEN - 2 + par
        pltpu.make_async_copy(bufs[par], out_hbm.at[t, d],
                              sem_s[par]).wait()


@jax.jit
def _embed(xt, tokT, pos_table):
    mesh = plsc.VectorSubcoreMesh(core_axis_name="c", subcore_axis_name="s")
    return pl.kernel(
        _body,
        out_type=jax.ShapeDtypeStruct((MAXLEN, D, BATCH), jnp.float32),
        mesh=mesh,
        scratch_types=[
            pltpu.VMEM((VOCAB,), jnp.float32),
            pltpu.VMEM((BATCH,), jnp.int32),
            pltpu.VMEM((BATCH,), jnp.int32),
            pltpu.VMEM((BATCH,), jnp.float32),
            pltpu.VMEM((BATCH,), jnp.float32),
            pltpu.VMEM((MAXLEN, D), jnp.float32),
            pltpu.SemaphoreType.DMA,
            pltpu.SemaphoreType.DMA,
            pltpu.SemaphoreType.DMA,
            pltpu.SemaphoreType.DMA,
        ],
        compiler_params=pltpu.CompilerParams(use_tc_tiling_on_sc=False,
                                             needs_layout_passes=False),
    )(xt, tokT, pos_table)


def kernel(x, token_table, pos_table):
    xt = jnp.swapaxes(x, 0, 1).astype(jnp.int32)
    tokT = jnp.swapaxes(token_table, 0, 1)
    out_tdb = _embed(xt, tokT, pos_table)
    return jnp.transpose(out_tdb, (2, 0, 1))


# speedup vs baseline: 2.4124x; 1.4581x over previous
import jax
import jax.numpy as jnp
from jax import lax
from jax.experimental import pallas as pl
from jax.experimental.pallas import tpu as pltpu
from jax.experimental.pallas import tpu_sc as plsc

NC = 2
NS = 16
NW = NC * NS

VOCAB = 100000
MAXLEN = 200
D = 32
BATCH = 4096
GROUPS = BATCH // 16


def _body(xt_hbm, tokT_hbm, pos_hbm, out_hbm, row_v, idx0, idx1,
          buf0, buf1, pos_v, sem_i0, sem_i1, sem_s0, sem_s1):
    d = lax.axis_index("s") * NC + lax.axis_index("c")

    idx = (idx0, idx1)
    bufs = (buf0, buf1)
    sem_i = (sem_i0, sem_i1)
    sem_s = (sem_s0, sem_s1)

    pltpu.sync_copy(tokT_hbm.at[d], row_v)
    pltpu.sync_copy(pos_hbm, pos_v)

    zeros = lax.iota(jnp.int32, 16) * 0

    pltpu.async_copy(xt_hbm.at[0], idx0, sem_i0)

    def pair_body(j, _):
        for par in range(2):
            t = 2 * j + par
            pltpu.make_async_copy(xt_hbm.at[t], idx[par], sem_i[par]).wait()

            @pl.when(t + 1 < MAXLEN)
            def _():
                pltpu.async_copy(xt_hbm.at[t + 1], idx[1 - par],
                                 sem_i[1 - par])

            pb = plsc.load_gather(pos_v, [zeros + t, zeros + d])

            @pl.when(t >= 2)
            def _():
                pltpu.make_async_copy(bufs[par], out_hbm.at[t - 2, d],
                                      sem_s[par]).wait()

            src = idx[par]
            dst = bufs[par]

            @plsc.parallel_loop(0, GROUPS, unroll=8)
            def _(g):
                iv = src[pl.ds(g * 16, 16)]
                v = plsc.load_gather(row_v, [iv])
                dst[pl.ds(g * 16, 16)] = v + pb

            pltpu.async_copy(bufs[par], out_hbm.at[t, d], sem_s[par])
        return 0

    lax.fori_loop(0, MAXLEN // 2, pair_body, 0)

    for par in range(2):
        t = MAXLEN - 2 + par
        pltpu.make_async_copy(bufs[par], out_hbm.at[t, d],
                              sem_s[par]).wait()


@jax.jit
def _embed(xt, tokT, pos_table):
    mesh = plsc.VectorSubcoreMesh(core_axis_name="c", subcore_axis_name="s")
    return pl.kernel(
        _body,
        out_type=jax.ShapeDtypeStruct((MAXLEN, D, BATCH), jnp.float32),
        mesh=mesh,
        scratch_types=[
            pltpu.VMEM((VOCAB,), jnp.float32),
            pltpu.VMEM((BATCH,), jnp.int32),
            pltpu.VMEM((BATCH,), jnp.int32),
            pltpu.VMEM((BATCH,), jnp.float32),
            pltpu.VMEM((BATCH,), jnp.float32),
            pltpu.VMEM((MAXLEN, D), jnp.float32),
            pltpu.SemaphoreType.DMA,
            pltpu.SemaphoreType.DMA,
            pltpu.SemaphoreType.DMA,
            pltpu.SemaphoreType.DMA,
        ],
        compiler_params=pltpu.CompilerParams(use_tc_tiling_on_sc=False,
                                             needs_layout_passes=False),
    )(xt, tokT, pos_table)


def kernel(x, token_table, pos_table):
    xt = jnp.swapaxes(x, 0, 1).astype(jnp.int32)
    tokT = jnp.swapaxes(token_table, 0, 1)
    out_tdb = _embed(xt, tokT, pos_table)
    return jnp.transpose(out_tdb, (2, 0, 1))


# speedup vs baseline: 3.0214x; 1.2524x over previous
import jax
import jax.numpy as jnp
from jax import lax
from jax.experimental import pallas as pl
from jax.experimental.pallas import tpu as pltpu
from jax.experimental.pallas import tpu_sc as plsc

NC = 2
NS = 16
NW = NC * NS

VOCAB = 100000
MAXLEN = 200
D = 32
BATCH = 4096
GROUPS = BATCH // 16
NBUF = 3


def _body(xt_hbm, tokT_hbm, pos_hbm, out_hbm, row_v, idx0, idx1, idx2,
          buf0, buf1, buf2, pos_v, si0, si1, si2, ss0, ss1, ss2):
    d = lax.axis_index("s") * NC + lax.axis_index("c")

    idx = (idx0, idx1, idx2)
    bufs = (buf0, buf1, buf2)
    sem_i = (si0, si1, si2)
    sem_s = (ss0, ss1, ss2)

    pltpu.sync_copy(tokT_hbm.at[d], row_v)
    pltpu.sync_copy(pos_hbm, pos_v)

    zeros = lax.iota(jnp.int32, 16) * 0

    for r in range(NBUF):
        pltpu.async_copy(xt_hbm.at[r], idx[r], sem_i[r])

    def step(t, r, drain, refire):
        pltpu.make_async_copy(xt_hbm.at[t], idx[r], sem_i[r]).wait()

        pb = plsc.load_gather(pos_v, [zeros + t, zeros + d])

        if drain:
            pltpu.make_async_copy(bufs[r], out_hbm.at[t - NBUF, d],
                                  sem_s[r]).wait()

        src = idx[r]
        dst = bufs[r]

        @plsc.parallel_loop(0, GROUPS, unroll=16)
        def _(g):
            iv = src[pl.ds(g * 16, 16)]
            v = plsc.load_gather(row_v, [iv])
            dst[pl.ds(g * 16, 16)] = v + pb

        pltpu.async_copy(bufs[r], out_hbm.at[t, d], sem_s[r])

        if refire:
            @pl.when(t + NBUF < MAXLEN)
            def _():
                pltpu.async_copy(xt_hbm.at[t + NBUF], idx[r], sem_i[r])

    for r in range(NBUF):
        step(r, r, drain=False, refire=True)

    def ring_body(j, _):
        for r in range(NBUF):
            step(NBUF * j + r, r, drain=True, refire=True)
        return 0

    lax.fori_loop(1, 66, ring_body, 0)

    for t in range(198, MAXLEN):
        step(t, t % NBUF, drain=True, refire=False)

    for t in range(MAXLEN - NBUF, MAXLEN):
        pltpu.make_async_copy(bufs[t % NBUF], out_hbm.at[t, d],
                              sem_s[t % NBUF]).wait()


@jax.jit
def _embed(xt, tokT, pos_table):
    mesh = plsc.VectorSubcoreMesh(core_axis_name="c", subcore_axis_name="s")
    return pl.kernel(
        _body,
        out_type=jax.ShapeDtypeStruct((MAXLEN, D, BATCH), jnp.float32),
        mesh=mesh,
        scratch_types=[
            pltpu.VMEM((VOCAB,), jnp.float32),
            pltpu.VMEM((BATCH,), jnp.int32),
            pltpu.VMEM((BATCH,), jnp.int32),
            pltpu.VMEM((BATCH,), jnp.int32),
            pltpu.VMEM((BATCH,), jnp.float32),
            pltpu.VMEM((BATCH,), jnp.float32),
            pltpu.VMEM((BATCH,), jnp.float32),
            pltpu.VMEM((MAXLEN, D), jnp.float32),
            pltpu.SemaphoreType.DMA,
            pltpu.SemaphoreType.DMA,
            pltpu.SemaphoreType.DMA,
            pltpu.SemaphoreType.DMA,
            pltpu.SemaphoreType.DMA,
            pltpu.SemaphoreType.DMA,
        ],
        compiler_params=pltpu.CompilerParams(use_tc_tiling_on_sc=False,
                                             needs_layout_passes=False),
    )(xt, tokT, pos_table)


def kernel(x, token_table, pos_table):
    xt = jnp.swapaxes(x, 0, 1).astype(jnp.int32)
    tokT = jnp.swapaxes(token_table, 0, 1)
    out_tdb = _embed(xt, tokT, pos_table)
    return jnp.transpose(out_tdb, (2, 0, 1))
